# baseline (speedup 1.0000x reference)
import functools

import numpy as np
import jax
import jax.numpy as jnp
from jax import lax
from jax.experimental import pallas as pl
from jax.experimental.pallas import tpu as pltpu
from jax.experimental.pallas import tpu_sc as plsc

N_LEVELS = 16
LOG2_HASHMAP_SIZE = 19
TBL = 1 << LOG2_HASHMAP_SIZE
MASK = TBL - 1
BASE_RES = 16.0
FINEST_RES = 512.0

NC = 2
NS = 16
NW = NC * NS

GB = 512

_B = np.exp((np.log(FINEST_RES) - np.log(BASE_RES)) / (N_LEVELS - 1))
RES = [float(np.floor(BASE_RES * (_B ** i))) for i in range(N_LEVELS)]
GS = [float(np.float32(1.0) / np.float32(r)) for r in RES]

P1 = np.uint32(2654435761).astype(np.int32)
P2 = np.uint32(805459861).astype(np.int32)


def _sc_body(C, xt, tab0, tab1, out, xbuf, wb0, wb1, idx0, idx1, emb0, emb1,
             obuf, sem0, sem1):
    NPTS = xt.shape[0] // 3
    PW = NPTS // NW
    GROUPS = C // 16
    CHUNKS = PW // C
    NB = (8 * C) // GB
    wid = lax.axis_index("s") * NC + lax.axis_index("c")

    def phase_a(l, idxb, wb):
        gs = jnp.float32(GS[l])
        lofs = np.int32(l * TBL)

        def g_body(g, c):
            gb16 = pl.multiple_of(g * 16, 16)
            p = [xbuf[pl.ds(gb16 + d * C, 16)] for d in range(3)]
            bls = []
            for d in range(3):
                xc = jnp.minimum(jnp.maximum(p[d], jnp.float32(0.0)),
                                 jnp.float32(1.0))
                pos = xc / gs
                bl = pos.astype(jnp.int32)
                blf = bl.astype(jnp.float32)
                vmin = blf * gs
                vmax = vmin + gs
                wb[pl.ds(gb16 + d * C, 16)] = (p[d] - vmin) / (vmax - vmin)
                bls.append(bl)
            a0 = bls[0]
            a1 = a0 + np.int32(1)
            b0 = bls[1] * P1
            b1 = b0 + P1
            c0 = bls[2] * P2
            c1 = c0 + P2
            tij = [a0 ^ b0, a0 ^ b1, a1 ^ b0, a1 ^ b1]
            for i in range(2):
                for j in range(2):
                    t = tij[i * 2 + j]
                    for k in range(2):
                        cc = i * 4 + j * 2 + k
                        h = ((t ^ (c0 if k == 0 else c1)) & MASK) + lofs
                        idxb[pl.ds(gb16 + cc * C, 16)] = h
            return c

        lax.fori_loop(0, GROUPS, g_body, 0)

    def fire(idxb, embb, sem):
        if NB == 1:
            pltpu.async_copy(tab0.at[idxb], embb.at[pl.ds(0, 8 * C)], sem)
            pltpu.async_copy(tab1.at[idxb], embb.at[pl.ds(8 * C, 8 * C)], sem)
            return

        def j_body(j, c):
            jb = pl.multiple_of(j * GB, GB)
            ixs = idxb.at[pl.ds(jb, GB)]
            pltpu.async_copy(tab0.at[ixs], embb.at[pl.ds(jb, GB)], sem)
            pltpu.async_copy(tab1.at[ixs], embb.at[pl.ds(8 * C + jb, GB)], sem)
            return c

        lax.fori_loop(0, NB, j_body, 0)

    def drain(idxb, embb, sem):
        pltpu.make_async_copy(tab0.at[idxb], embb.at[pl.ds(0, 8 * C)],
                              sem).wait()
        pltpu.make_async_copy(tab1.at[idxb], embb.at[pl.ds(8 * C, 8 * C)],
                              sem).wait()

    def phase_b(l, wb, embb):
        def g_body(g, c):
            gb16 = pl.multiple_of(g * 16, 16)
            w0 = wb[pl.ds(gb16 + 0 * C, 16)]
            w1 = wb[pl.ds(gb16 + 1 * C, 16)]
            w2 = wb[pl.ds(gb16 + 2 * C, 16)]
            u0 = jnp.float32(1.0) - w0
            u1 = jnp.float32(1.0) - w1
            u2 = jnp.float32(1.0) - w2
            for f in range(2):
                e0, e1, e2, e3, e4, e5, e6, e7 = [
                    embb[pl.ds(gb16 + (f * 8 + cc) * C, 16)]
                    for cc in range(8)]
                c00 = e0 * u0 + e4 * w0
                c01 = e1 * u0 + e5 * w0
                c10 = e2 * u0 + e6 * w0
                c11 = e3 * u0 + e7 * w0
                d0 = c00 * u1 + c10 * w1
                d1 = c01 * u1 + c11 * w1
                ob = ((gb16 // 128) * 32 + 2 * l + f) * 128 + gb16 % 128
                obuf[pl.ds(ob, 16)] = d0 * u2 + d1 * w2
            return c

        lax.fori_loop(0, GROUPS, g_body, 0)

    bufs = [(idx0, wb0, emb0, sem0), (idx1, wb1, emb1, sem1)]

    def chunk(ch, c):
        base = wid * PW + ch * C
        for r in range(3):
            pltpu.sync_copy(xt.at[pl.ds(r * NPTS + base, C)],
                            xbuf.at[pl.ds(r * C, C)])
        phase_a(0, idx0, wb0)
        fire(idx0, emb0, sem0)
        for l in range(N_LEVELS):
            if l + 1 < N_LEVELS:
                ib, wbn, eb, sm = bufs[(l + 1) % 2]
                phase_a(l + 1, ib, wbn)
                fire(ib, eb, sm)
            ib, wbc, eb, sm = bufs[l % 2]
            drain(ib, eb, sm)
            phase_b(l, wbc, eb)
        for b in range(C // 128):
            pltpu.sync_copy(obuf.at[pl.ds(b * 4096, 4096)],
                            out.at[pl.ds(base * 32 + b * 4096, 4096)])
        return c

    lax.fori_loop(0, CHUNKS, chunk, 0)


def _make_sc_call(n_points, C):
    mesh = plsc.VectorSubcoreMesh(core_axis_name="c", subcore_axis_name="s")
    return pl.kernel(
        functools.partial(_sc_body, C),
        out_type=jax.ShapeDtypeStruct((n_points * 2 * N_LEVELS,), jnp.float32),
        mesh=mesh,
        scratch_types=[
            pltpu.VMEM((3 * C,), jnp.float32),
            pltpu.VMEM((3 * C,), jnp.float32),
            pltpu.VMEM((3 * C,), jnp.float32),
            pltpu.VMEM((8 * C,), jnp.int32),
            pltpu.VMEM((8 * C,), jnp.int32),
            pltpu.VMEM((16 * C,), jnp.float32),
            pltpu.VMEM((16 * C,), jnp.float32),
            pltpu.VMEM((C * 32,), jnp.float32),
            pltpu.SemaphoreType.DMA,
            pltpu.SemaphoreType.DMA,
        ],
    )


def kernel(x, tables):
    n = x.shape[0]
    pw = n // NW
    C = min(1024, pw)
    xt = x.T.reshape(3 * n)
    tf = tables.transpose(2, 0, 1).reshape(2, N_LEVELS * TBL)
    out_flat = _make_sc_call(n, C)(xt, tf[0], tf[1])
    out = out_flat.reshape(n // 128, 2 * N_LEVELS, 128)
    out = out.transpose(0, 2, 1).reshape(n, 2 * N_LEVELS)
    keep = jnp.logical_not(jnp.any(jnp.isnan(x), axis=-1))
    return out, keep


# speedup vs baseline: 1.0863x; 1.0863x over previous
import functools

import numpy as np
import jax
import jax.numpy as jnp
from jax import lax
from jax.experimental import pallas as pl
from jax.experimental.pallas import tpu as pltpu
from jax.experimental.pallas import tpu_sc as plsc

N_LEVELS = 16
LOG2_HASHMAP_SIZE = 19
TBL = 1 << LOG2_HASHMAP_SIZE
MASK = TBL - 1
BASE_RES = 16.0
FINEST_RES = 512.0

NC = 2
NS = 16
NW = NC * NS

GB = 512

_B = np.exp((np.log(FINEST_RES) - np.log(BASE_RES)) / (N_LEVELS - 1))
RES = [float(np.floor(BASE_RES * (_B ** i))) for i in range(N_LEVELS)]
GS = [float(np.float32(1.0) / np.float32(r)) for r in RES]

P1 = np.uint32(2654435761).astype(np.int32)
P2 = np.uint32(805459861).astype(np.int32)


def _sc_body(C, xt, tab0, tab1, out, xbuf, wb0, wb1, idx0, idx1, emb0, emb1,
             obuf, grid0, grid1, sem0, sem1):
    NPTS = xt.shape[0] // 3
    PW = NPTS // NW
    GROUPS = C // 16
    CHUNKS = PW // C
    NB = (8 * C) // GB
    sid = lax.axis_index("s")
    wid = sid * NC + lax.axis_index("c")

    MST = TBL // NS
    mofs = sid * MST
    pltpu.sync_copy(tab0.at[pl.ds(mofs, MST)], grid0.at[pl.ds(mofs, MST)])
    pltpu.sync_copy(tab1.at[pl.ds(mofs, MST)], grid1.at[pl.ds(mofs, MST)])
    plsc.subcore_barrier()

    def phase_a(l, idxb, wb):
        gs = jnp.float32(GS[l])
        lofs = np.int32(l * TBL)

        def g_body(g, c):
            gb16 = pl.multiple_of(g * 16, 16)
            p = [xbuf[pl.ds(gb16 + d * C, 16)] for d in range(3)]
            bls = []
            for d in range(3):
                xc = jnp.minimum(jnp.maximum(p[d], jnp.float32(0.0)),
                                 jnp.float32(1.0))
                pos = xc / gs
                bl = pos.astype(jnp.int32)
                blf = bl.astype(jnp.float32)
                vmin = blf * gs
                vmax = vmin + gs
                wb[pl.ds(gb16 + d * C, 16)] = (p[d] - vmin) / (vmax - vmin)
                bls.append(bl)
            a0 = bls[0]
            a1 = a0 + np.int32(1)
            b0 = bls[1] * P1
            b1 = b0 + P1
            c0 = bls[2] * P2
            c1 = c0 + P2
            tij = [a0 ^ b0, a0 ^ b1, a1 ^ b0, a1 ^ b1]
            for i in range(2):
                for j in range(2):
                    t = tij[i * 2 + j]
                    for k in range(2):
                        cc = i * 4 + j * 2 + k
                        h = ((t ^ (c0 if k == 0 else c1)) & MASK) + lofs
                        idxb[pl.ds(gb16 + cc * C, 16)] = h
            return c

        lax.fori_loop(0, GROUPS, g_body, 0)

    def fire(l, idxb, embb, sem):
        s0, s1 = (grid0, grid1) if l == 0 else (tab0, tab1)

        def j_body(j, c):
            jb = pl.multiple_of(j * GB, GB)
            ixs = idxb.at[pl.ds(jb, GB)]
            pltpu.async_copy(s0.at[ixs], embb.at[pl.ds(jb, GB)], sem)
            pltpu.async_copy(s1.at[ixs], embb.at[pl.ds(8 * C + jb, GB)], sem)
            return c

        lax.fori_loop(0, NB, j_body, 0)

    def drain(idxb, embb, sem):
        pltpu.make_async_copy(tab0.at[idxb], embb.at[pl.ds(0, 8 * C)],
                              sem).wait()
        pltpu.make_async_copy(tab1.at[idxb], embb.at[pl.ds(8 * C, 8 * C)],
                              sem).wait()

    def phase_b(l, wb, embb):
        def g_body(g, c):
            gb16 = pl.multiple_of(g * 16, 16)
            w0 = wb[pl.ds(gb16 + 0 * C, 16)]
            w1 = wb[pl.ds(gb16 + 1 * C, 16)]
            w2 = wb[pl.ds(gb16 + 2 * C, 16)]
            u0 = jnp.float32(1.0) - w0
            u1 = jnp.float32(1.0) - w1
            u2 = jnp.float32(1.0) - w2
            for f in range(2):
                e0, e1, e2, e3, e4, e5, e6, e7 = [
                    embb[pl.ds(gb16 + (f * 8 + cc) * C, 16)]
                    for cc in range(8)]
                c00 = e0 * u0 + e4 * w0
                c01 = e1 * u0 + e5 * w0
                c10 = e2 * u0 + e6 * w0
                c11 = e3 * u0 + e7 * w0
                d0 = c00 * u1 + c10 * w1
                d1 = c01 * u1 + c11 * w1
                ob = ((gb16 // 128) * 32 + 2 * l + f) * 128 + gb16 % 128
                obuf[pl.ds(ob, 16)] = d0 * u2 + d1 * w2
            return c

        lax.fori_loop(0, GROUPS, g_body, 0)

    bufs = [(idx0, wb0, emb0, sem0), (idx1, wb1, emb1, sem1)]

    def chunk(ch, c):
        base = wid * PW + ch * C
        for r in range(3):
            pltpu.sync_copy(xt.at[pl.ds(r * NPTS + base, C)],
                            xbuf.at[pl.ds(r * C, C)])
        phase_a(0, idx0, wb0)
        fire(0, idx0, emb0, sem0)
        for l in range(N_LEVELS):
            if l + 1 < N_LEVELS:
                ib, wbn, eb, sm = bufs[(l + 1) % 2]
                phase_a(l + 1, ib, wbn)
                fire(l + 1, ib, eb, sm)
            ib, wbc, eb, sm = bufs[l % 2]
            drain(ib, eb, sm)
            phase_b(l, wbc, eb)
        for b in range(C // 128):
            pltpu.sync_copy(obuf.at[pl.ds(b * 4096, 4096)],
                            out.at[pl.ds(base * 32 + b * 4096, 4096)])
        return c

    lax.fori_loop(0, CHUNKS, chunk, 0)


def _make_sc_call(n_points, C):
    mesh = plsc.VectorSubcoreMesh(core_axis_name="c", subcore_axis_name="s")
    return pl.kernel(
        functools.partial(_sc_body, C),
        out_type=jax.ShapeDtypeStruct((n_points * 2 * N_LEVELS,), jnp.float32),
        mesh=mesh,
        scratch_types=[
            pltpu.VMEM((3 * C,), jnp.float32),
            pltpu.VMEM((3 * C,), jnp.float32),
            pltpu.VMEM((3 * C,), jnp.float32),
            pltpu.VMEM((8 * C,), jnp.int32),
            pltpu.VMEM((8 * C,), jnp.int32),
            pltpu.VMEM((16 * C,), jnp.float32),
            pltpu.VMEM((16 * C,), jnp.float32),
            pltpu.VMEM((C * 32,), jnp.float32),
            pltpu.VMEM_SHARED((TBL,), jnp.float32),
            pltpu.VMEM_SHARED((TBL,), jnp.float32),
            pltpu.SemaphoreType.DMA,
            pltpu.SemaphoreType.DMA,
        ],
    )


def kernel(x, tables):
    n = x.shape[0]
    pw = n // NW
    C = min(512, pw)
    xt = x.T.reshape(3 * n)
    tf = tables.transpose(2, 0, 1).reshape(2, N_LEVELS * TBL)
    out_flat = _make_sc_call(n, C)(xt, tf[0], tf[1])
    out = out_flat.reshape(n // 128, 2 * N_LEVELS, 128)
    out = out.transpose(0, 2, 1).reshape(n, 2 * N_LEVELS)
    keep = jnp.logical_not(jnp.any(jnp.isnan(x), axis=-1))
    return out, keep
